# Initial kernel scaffold; baseline (speedup 1.0000x reference)
import jax
import jax.numpy as jnp
from jax.experimental import pallas as pl


def kernel(inputs, labels, adj, W_enc_g, b_enc_g, W_enc_c, b_enc_c, W_dec_g, b_dec_g, W_dec_c, b_dec_c, Wp, bp):
    raise NotImplementedError("write your pallas kernel here")


# speedup vs baseline: 9.6419x; 9.6419x over previous
import functools

import jax
import jax.numpy as jnp
from jax.experimental import pallas as pl

N = 325
NP = 384
B = 16
U = 128
SEQ = 12
HOR = 12
M = 5

_f32 = jnp.float32


def _mmT(A, X):
    return jax.lax.dot_general(A, X, (((0,), (0,)), ((), ())),
                               preferred_element_type=_f32)


def _mm(A, X):
    return jax.lax.dot_general(A, X, (((1,), (0,)), ((), ())),
                               preferred_element_type=_f32)


def _dcrnn_body(adj_ref, xseq_ref,
                wgs_e_ref, kg_e_ref, bg_e_ref, wcs_e_ref, kc_e_ref, bc_e_ref,
                wgs_d_ref, kg_d_ref, bg_d_ref, wcs_d_ref, kc_d_ref, bc_d_ref,
                wpsel_ref, bp_ref, out_ref):
    A = adj_ref[...]
    d = jnp.sum(A, axis=1, keepdims=True)
    dinv = jnp.where(d > 0, 1.0 / d, 0.0)
    e = jnp.sum(A, axis=0, keepdims=True)
    einv = jnp.where(e > 0, 1.0 / e, 0.0)
    B1 = dinv * A
    B2 = A * einv

    def diffuse(X):
        x1a = _mmT(B1, X)
        x2a = 2.0 * _mmT(B1, x1a) - X
        x1b = _mm(B2, X)
        x2b = 2.0 * _mm(B2, x1b) - X
        return (X, x1a, x2a, x1b, x2b)

    def xi_contrib(xi_m, k_ref, P):
        acc = jnp.dot(xi_m[0], k_ref[0], preferred_element_type=_f32)
        for m in range(1, M):
            acc = acc + jnp.dot(xi_m[m], k_ref[m], preferred_element_type=_f32)
        return acc.reshape(NP * B, P)

    def gconv(xic, h_m6, ws_ref, b_ref, P):
        acc = xic + jnp.broadcast_to(b_ref[...], (NP * B, P))
        for m in range(M):
            acc = acc + jnp.dot(h_m6[m], ws_ref[m], preferred_element_type=_f32)
        return acc

    def cell(xi_m, h2d, h6, wgs, kg, bg, wcs, kc, bc):
        h_m = diffuse(h2d)
        h_m6 = [h6] + [x.reshape(NP * B, U) for x in h_m[1:]]
        xg = xi_contrib(xi_m, kg, 2 * U)
        G = jax.nn.sigmoid(gconv(xg, h_m6, wgs, bg, 2 * U))
        r = G[:, :U]
        u = G[:, U:]
        rh6 = r * h6
        c_m = diffuse(rh6.reshape(NP, B * U))
        c_m6 = [rh6] + [x.reshape(NP * B, U) for x in c_m[1:]]
        xc = xi_contrib(xi_m, kc, U)
        C = jnp.tanh(gconv(xc, c_m6, wcs, bc, U))
        h_new6 = u * h6 + (1.0 - u) * C
        return h_new6.reshape(NP, B * U), h_new6

    h2d = jnp.zeros((NP, B * U), _f32)
    h6 = jnp.zeros((NP * B, U), _f32)
    for t in range(SEQ):
        xi = xseq_ref[t]
        xi_m = diffuse(xi)
        h2d, h6 = cell(xi_m, h2d, h6,
                       wgs_e_ref, kg_e_ref, bg_e_ref,
                       wcs_e_ref, kc_e_ref, bc_e_ref)

    dec_in = jnp.zeros((NP, B), _f32)
    bp0 = bp_ref[0, 0]
    for t in range(HOR):
        xi_m = diffuse(dec_in)
        h2d, h6 = cell(xi_m, h2d, h6,
                       wgs_d_ref, kg_d_ref, bg_d_ref,
                       wcs_d_ref, kc_d_ref, bc_d_ref)
        dec_in = jnp.dot(h2d, wpsel_ref[...], preferred_element_type=_f32) + bp0
        out_ref[t] = dec_in


@jax.jit
def _run(adj_p, xseq, wgs_e, kg_e, bg_e, wcs_e, kc_e, bc_e,
         wgs_d, kg_d, bg_d, wcs_d, kc_d, bc_d, wpsel, bp2):
    return pl.pallas_call(
        _dcrnn_body,
        out_shape=jax.ShapeDtypeStruct((HOR, NP, B), _f32),
    )(adj_p, xseq, wgs_e, kg_e, bg_e, wcs_e, kc_e, bc_e,
      wgs_d, kg_d, bg_d, wcs_d, kc_d, bc_d, wpsel, bp2)


def _split_w(W, P):
    Wr = W.reshape(1 + U, M, P)
    Wi = jnp.transpose(Wr[0:1], (1, 0, 2))
    Ws = jnp.transpose(Wr[1:], (1, 0, 2))
    eye = jnp.eye(B, dtype=W.dtype)
    K = (eye[None, :, :, None] * Wi[:, :, None, :]).reshape(M, B, B * P)
    return Ws, K


def kernel(inputs, labels, adj, W_enc_g, b_enc_g, W_enc_c, b_enc_c,
           W_dec_g, b_dec_g, W_dec_c, b_dec_c, Wp, bp):
    del labels
    adj_p = jnp.zeros((NP, NP), _f32).at[:N, :N].set(adj)
    xseq = jnp.zeros((SEQ, NP, B), _f32).at[:, :N, :].set(
        jnp.transpose(inputs, (2, 1, 0)))
    wgs_e, kg_e = _split_w(W_enc_g, 2 * U)
    wcs_e, kc_e = _split_w(W_enc_c, U)
    wgs_d, kg_d = _split_w(W_dec_g, 2 * U)
    wcs_d, kc_d = _split_w(W_dec_c, U)
    wpsel = (jnp.eye(B, dtype=_f32)[:, None, :] * Wp[None, :, 0:1]).reshape(B * U, B)
    out = _run(adj_p, xseq,
               wgs_e, kg_e, b_enc_g.reshape(1, 2 * U),
               wcs_e, kc_e, b_enc_c.reshape(1, U),
               wgs_d, kg_d, b_dec_g.reshape(1, 2 * U),
               wcs_d, kc_d, b_dec_c.reshape(1, U),
               wpsel, bp.reshape(1, 1))
    return jnp.transpose(out[:, :N, :], (2, 1, 0))
